# Initial kernel scaffold; baseline (speedup 1.0000x reference)
import jax
import jax.numpy as jnp
from jax.experimental import pallas as pl


def kernel(length_q, length_k, embeddings_table_v, embeddings_table_h):
    raise NotImplementedError("write your pallas kernel here")


# speedup vs baseline: 11.7157x; 11.7157x over previous
import functools

import jax
import jax.numpy as jnp
from jax import lax
from jax.experimental import pallas as pl
from jax.experimental.pallas import tpu as pltpu, tpu_sc as plsc

L = 1025
S = 32
NB = 32
D = 64
NLANE = 16
NV = D // NLANE


def _body(tv_hbm, th_hbm, evi_hbm, ehi_hbm, out_hbm,
          ivx, ihx, tvx, thx, buf, sem):
    c = lax.axis_index("c")
    s = lax.axis_index("s")
    wid = s * 2 + c

    pltpu.sync_copy(evi_hbm, ivx)
    pltpu.sync_copy(ehi_hbm, ihx)
    pltpu.async_copy(tv_hbm.at[ivx], tvx, sem).wait()
    pltpu.async_copy(th_hbm.at[ihx], thx, sem).wait()

    cst = [tvx[63, pl.ds(k * NLANE, NLANE)] + thx[63, pl.ds(k * NLANE, NLANE)]
           for k in range(NV)]

    def row_body(ri, carry):
        for k in range(NV):
            buf[0, pl.ds(k * NLANE, NLANE)] = cst[k]
        for bj in range(NB):
            dv = 31 + bj - wid
            vb = [tvx[dv, pl.ds(k * NLANE, NLANE)] for k in range(NV)]
            base = 1 + bj * S

            def rj_body(rj, carry2):
                dh = 31 + rj - ri
                for k in range(NV):
                    sl = pl.ds(k * NLANE, NLANE)
                    buf[base + rj, sl] = vb[k] + thx[dh, sl]
                return carry2

            lax.fori_loop(0, S, rj_body, 0)
        row_i = 1 + S * wid + ri
        pltpu.sync_copy(buf, out_hbm.at[pl.ds(row_i * L, L)])
        return carry

    lax.fori_loop(0, S, row_body, 0)

    @pl.when(wid == 0)
    def _():
        def fill_body(r, carry):
            for k in range(NV):
                buf[r, pl.ds(k * NLANE, NLANE)] = cst[k]
            return carry

        lax.fori_loop(0, L, fill_body, 0)
        pltpu.sync_copy(buf, out_hbm.at[pl.ds(0, L)])


@jax.jit
def kernel(length_q, length_k, embeddings_table_v, embeddings_table_h):
    zq = (jnp.asarray(length_q) - L).astype(jnp.int32)
    zk = (jnp.asarray(length_k) - L).astype(jnp.int32)
    d = jnp.arange(64, dtype=jnp.int32) - 31
    evi = jnp.clip(d, -14, 14) + 15 + zq
    ehi = jnp.clip(d, -14, 14) + 15 + zk
    evi = evi.at[63].set(zq)
    ehi = ehi.at[63].set(zk)

    mesh = plsc.VectorSubcoreMesh(core_axis_name="c", subcore_axis_name="s")
    run = pl.kernel(
        _body,
        out_type=jax.ShapeDtypeStruct((L * L, D), jnp.float32),
        mesh=mesh,
        compiler_params=pltpu.CompilerParams(use_tc_tiling_on_sc=False),
        scratch_types=[
            pltpu.VMEM((64,), jnp.int32),
            pltpu.VMEM((64,), jnp.int32),
            pltpu.VMEM((64, D), jnp.float32),
            pltpu.VMEM((64, D), jnp.float32),
            pltpu.VMEM((L, D), jnp.float32),
            pltpu.SemaphoreType.DMA,
        ],
    )
    flat = run(embeddings_table_v, embeddings_table_h, evi, ehi)
    return flat.reshape(L, L, D)


# speedup vs baseline: 20.3746x; 1.7391x over previous
import jax
import jax.numpy as jnp
from jax import lax
from jax.experimental import pallas as pl
from jax.experimental.pallas import tpu as pltpu, tpu_sc as plsc

L = 1025
S = 32
NB = 32
D = 64
NLANE = 16
NV = D // NLANE
NW = 61
CROWS = 24
MAXQ = 14


def _sl(k):
    return pl.ds(k * NLANE, NLANE)


def _body(tv_hbm, th_hbm, evi_hbm, ehi_hbm, out_hbm,
          iv, ih, tvc, thc, wnd, cbuf, sem):
    cc = lax.axis_index("c")
    ss = lax.axis_index("s")
    ri = ss * 2 + cc

    pltpu.sync_copy(evi_hbm, iv)
    pltpu.sync_copy(ehi_hbm, ih)
    pltpu.async_copy(tv_hbm.at[iv], tvc, sem).wait()
    pltpu.async_copy(th_hbm.at[ih], thc, sem).wait()

    def wnd_body(w, carry):
        cv = jnp.clip(w - 30, -MAXQ, MAXQ) + MAXQ
        vb = [tvc[cv, _sl(k)] for k in range(NV)]
        base = w * S
        for rj in range(S):
            ce = jnp.clip(rj - ri, -MAXQ, MAXQ) + MAXQ
            for k in range(NV):
                wnd[base + rj, _sl(k)] = vb[k] + thc[ce, _sl(k)]
        return carry

    lax.fori_loop(0, NW, wnd_body, 0)

    cst = [tvc[29, _sl(k)] + thc[29, _sl(k)] for k in range(NV)]

    def cb_body(r, carry):
        for k in range(NV):
            cbuf[r, _sl(k)] = cst[k]
        return carry

    lax.fori_loop(0, CROWS, cb_body, 0)

    descs = []

    def push(d):
        descs.append(d)
        if len(descs) > MAXQ:
            descs.pop(0).wait()

    for bi in range(NB):
        base = (1 + S * bi + ri) * L
        push(pltpu.async_copy(cbuf.at[pl.ds(0, 1)],
                              out_hbm.at[pl.ds(base, 1)], sem))
        if bi == 0:
            push(pltpu.async_copy(wnd.at[pl.ds(30 * S, 31 * S)],
                                  out_hbm.at[pl.ds(base + 1, 31 * S)], sem))
            push(pltpu.async_copy(wnd.at[pl.ds(60 * S, S)],
                                  out_hbm.at[pl.ds(base + 1 + 31 * S, S)], sem))
        elif bi == 31:
            push(pltpu.async_copy(wnd.at[pl.ds(0, S)],
                                  out_hbm.at[pl.ds(base + 1, S)], sem))
            push(pltpu.async_copy(wnd.at[pl.ds(0, 31 * S)],
                                  out_hbm.at[pl.ds(base + 1 + S, 31 * S)], sem))
        else:
            push(pltpu.async_copy(wnd.at[pl.ds((30 - bi) * S, NB * S)],
                                  out_hbm.at[pl.ds(base + 1, NB * S)], sem))
    for d in descs:
        d.wait()

    @pl.when(ri == 0)
    def _():
        descs0 = []

        def push0(d):
            descs0.append(d)
            if len(descs0) > MAXQ:
                descs0.pop(0).wait()

        full = L // CROWS
        for t in range(full):
            push0(pltpu.async_copy(cbuf.at[pl.ds(0, CROWS)],
                                   out_hbm.at[pl.ds(t * CROWS, CROWS)], sem))
        rem = L - full * CROWS
        push0(pltpu.async_copy(cbuf.at[pl.ds(0, rem)],
                               out_hbm.at[pl.ds(full * CROWS, rem)], sem))
        for d in descs0:
            d.wait()


@jax.jit
def kernel(length_q, length_k, embeddings_table_v, embeddings_table_h):
    zq = (jnp.asarray(length_q) - L).astype(jnp.int32)
    zk = (jnp.asarray(length_k) - L).astype(jnp.int32)
    t = jnp.arange(32, dtype=jnp.int32)
    evi = jnp.where(t < 29, t + 1, 0) + zq
    ehi = jnp.where(t < 29, t + 1, 0) + zk

    mesh = plsc.VectorSubcoreMesh(core_axis_name="c", subcore_axis_name="s")
    run = pl.kernel(
        _body,
        out_type=jax.ShapeDtypeStruct((L * L, D), jnp.float32),
        mesh=mesh,
        compiler_params=pltpu.CompilerParams(use_tc_tiling_on_sc=False),
        scratch_types=[
            pltpu.VMEM((32,), jnp.int32),
            pltpu.VMEM((32,), jnp.int32),
            pltpu.VMEM((32, D), jnp.float32),
            pltpu.VMEM((32, D), jnp.float32),
            pltpu.VMEM((NW * S, D), jnp.float32),
            pltpu.VMEM((CROWS, D), jnp.float32),
            pltpu.SemaphoreType.DMA,
        ],
    )
    flat = run(embeddings_table_v, embeddings_table_h, evi, ehi)
    return flat.reshape(L, L, D)


# speedup vs baseline: 20.4681x; 1.0046x over previous
import jax
import jax.numpy as jnp
from jax import lax
from jax.experimental import pallas as pl
from jax.experimental.pallas import tpu as pltpu, tpu_sc as plsc

L = 1025
S = 32
NB = 32
D = 64
NLANE = 16
NV = D // NLANE
NW = 61
CROWS = 24
MAXQ = 14


def _sl(k):
    return pl.ds(k * NLANE, NLANE)


def _body(tv_hbm, th_hbm, evi_hbm, ehi_hbm, out_hbm,
          iv, ih, tvc, thc, wnd, cbuf, sem):
    cc = lax.axis_index("c")
    ss = lax.axis_index("s")
    ri = ss * 2 + cc

    pltpu.sync_copy(evi_hbm, iv)
    pltpu.sync_copy(ehi_hbm, ih)
    pltpu.async_copy(tv_hbm.at[iv], tvc, sem).wait()
    pltpu.async_copy(th_hbm.at[ih], thc, sem).wait()

    def wnd_body(w, carry):
        cv = jnp.clip(w - 30, -MAXQ, MAXQ) + MAXQ
        vb = [tvc[cv, _sl(k)] for k in range(NV)]
        base = w * S
        for rj in range(S):
            ce = jnp.clip(rj - ri, -MAXQ, MAXQ) + MAXQ
            for k in range(NV):
                wnd[base + rj, _sl(k)] = vb[k] + thc[ce, _sl(k)]
        return carry

    lax.fori_loop(0, NW, wnd_body, 0)

    cst = [tvc[29, _sl(k)] + thc[29, _sl(k)] for k in range(NV)]

    def cb_body(r, carry):
        for k in range(NV):
            cbuf[r, _sl(k)] = cst[k]
        return carry

    lax.fori_loop(0, CROWS, cb_body, 0)

    descs = []

    def push(d):
        descs.append(d)
        if len(descs) > MAXQ:
            descs.pop(0).wait()

    for bi in range(NB):
        row = 1 + S * bi + ri
        push(pltpu.async_copy(cbuf.at[pl.ds(0, 1)],
                              out_hbm.at[row, pl.ds(0, 1)], sem))
        if bi == 0:
            push(pltpu.async_copy(wnd.at[pl.ds(30 * S, 31 * S)],
                                  out_hbm.at[row, pl.ds(1, 31 * S)], sem))
            push(pltpu.async_copy(wnd.at[pl.ds(60 * S, S)],
                                  out_hbm.at[row, pl.ds(1 + 31 * S, S)], sem))
        elif bi == 31:
            push(pltpu.async_copy(wnd.at[pl.ds(0, S)],
                                  out_hbm.at[row, pl.ds(1, S)], sem))
            push(pltpu.async_copy(wnd.at[pl.ds(0, 31 * S)],
                                  out_hbm.at[row, pl.ds(1 + S, 31 * S)], sem))
        else:
            push(pltpu.async_copy(wnd.at[pl.ds((30 - bi) * S, NB * S)],
                                  out_hbm.at[row, pl.ds(1, NB * S)], sem))
    for d in descs:
        d.wait()

    @pl.when(ri == 0)
    def _():
        descs0 = []

        def push0(d):
            descs0.append(d)
            if len(descs0) > MAXQ:
                descs0.pop(0).wait()

        full = L // CROWS
        for t in range(full):
            push0(pltpu.async_copy(cbuf.at[pl.ds(0, CROWS)],
                                   out_hbm.at[0, pl.ds(t * CROWS, CROWS)], sem))
        rem = L - full * CROWS
        push0(pltpu.async_copy(cbuf.at[pl.ds(0, rem)],
                               out_hbm.at[0, pl.ds(full * CROWS, rem)], sem))
        for d in descs0:
            d.wait()


@jax.jit
def kernel(length_q, length_k, embeddings_table_v, embeddings_table_h):
    zq = (jnp.asarray(length_q) - L).astype(jnp.int32)
    zk = (jnp.asarray(length_k) - L).astype(jnp.int32)
    t = jnp.arange(32, dtype=jnp.int32)
    evi = jnp.where(t < 29, t + 1, 0) + zq
    ehi = jnp.where(t < 29, t + 1, 0) + zk

    mesh = plsc.VectorSubcoreMesh(core_axis_name="c", subcore_axis_name="s")
    run = pl.kernel(
        _body,
        out_type=jax.ShapeDtypeStruct((L, L, D), jnp.float32),
        mesh=mesh,
        compiler_params=pltpu.CompilerParams(use_tc_tiling_on_sc=False),
        scratch_types=[
            pltpu.VMEM((32,), jnp.int32),
            pltpu.VMEM((32,), jnp.int32),
            pltpu.VMEM((32, D), jnp.float32),
            pltpu.VMEM((32, D), jnp.float32),
            pltpu.VMEM((NW * S, D), jnp.float32),
            pltpu.VMEM((CROWS, D), jnp.float32),
            pltpu.SemaphoreType.DMA,
        ],
    )
    return run(embeddings_table_v, embeddings_table_h, evi, ehi)


# speedup vs baseline: 30.0632x; 1.4688x over previous
import jax
import jax.numpy as jnp
from jax import lax
from jax.experimental import pallas as pl
from jax.experimental.pallas import tpu as pltpu, tpu_sc as plsc

L = 1025
S = 32
NB = 32
D = 64
NLANE = 16
NV = D // NLANE
CLIP = 14
WROWS = 1016
HALF = 512
CROWS = 24
NHEAD = 15
MAXQ = 14


def _sl(k):
    return pl.ds(k * NLANE, NLANE)


def _emit_plan(bi):
    plan = []
    lext = max(0, 32 * (bi - 15))
    j = 8
    while lext > 0:
        plan.append((j, 32, 'L', 0))
        j += 32
        lext -= 32
    j_end = min(1024, 479 + 32 * bi)
    q0 = j + 472 - 32 * bi
    q1 = j_end + 472 - 32 * bi
    if q0 <= min(q1, HALF - 1):
        n = min(q1, HALF - 1) - q0 + 1
        plan.append((j, n, 'A', q0))
        j += n
    if max(q0, HALF) <= q1:
        n = q1 - max(q0, HALF) + 1
        plan.append((j, n, 'B', max(q0, HALF) - HALF))
        j += n
    while j <= 1024:
        n = min(64, 1025 - j)
        plan.append((j, n, 'R', 440))
        j += n
    return plan


def _body(tv_hbm, th_hbm, evi_hbm, ehi_hbm, out_hbm,
          iv, ih, tvc, thc, wnd, head, cbuf, sem):
    cc = lax.axis_index("c")
    ss = lax.axis_index("s")
    ri = ss * 2 + cc

    pltpu.sync_copy(evi_hbm, iv)
    pltpu.sync_copy(ehi_hbm, ih)
    pltpu.async_copy(tv_hbm.at[iv], tvc, sem).wait()
    pltpu.async_copy(th_hbm.at[ih], thc, sem).wait()

    cst = [tvc[29, _sl(k)] + thc[29, _sl(k)] for k in range(NV)]

    def head_body(t, carry):
        cv = CLIP - jnp.minimum(t, CLIP)
        vb = [tvc[cv, _sl(k)] for k in range(NV)]
        base = t * 8
        for k in range(NV):
            head[base, _sl(k)] = cst[k]
        for u in range(7):
            ce = jnp.clip(u - ri, -CLIP, CLIP) + CLIP
            for k in range(NV):
                head[base + 1 + u, _sl(k)] = vb[k] + thc[ce, _sl(k)]
        return carry

    lax.fori_loop(0, NHEAD, head_body, 0)

    def cb_body(r, carry):
        for k in range(NV):
            cbuf[r, _sl(k)] = cst[k]
        return carry

    lax.fori_loop(0, CROWS, cb_body, 0)

    def mk_wnd_body(base):
        def wnd_body(q, carry):
            qq = q - 25
            cv = jnp.clip(lax.shift_right_arithmetic(qq, 5) - CLIP,
                          -CLIP, CLIP) + CLIP
            ce = jnp.clip(lax.bitwise_and(qq, 31) - ri, -CLIP, CLIP) + CLIP
            r = q - base
            for k in range(NV):
                wnd[r, _sl(k)] = tvc[cv, _sl(k)] + thc[ce, _sl(k)]
            return carry
        return wnd_body

    descs = []

    def push(d):
        descs.append(d)
        if len(descs) > MAXQ:
            descs.pop(0).wait()

    def drain():
        while descs:
            descs.pop(0).wait()

    plans = [_emit_plan(bi) for bi in range(NB)]

    for phase, (base, hi) in enumerate([(0, HALF), (HALF, WROWS)]):
        lax.fori_loop(base, hi, mk_wnd_body(base), 0)
        if phase == 0:
            for bi in range(NB):
                row = 1 + S * bi + ri
                push(pltpu.async_copy(
                    head.at[pl.ds(8 * min(bi, CLIP), 8)],
                    out_hbm.at[row, pl.ds(0, 8)], sem))
        for bi in range(NB):
            row = 1 + S * bi + ri
            for (dst_j, n, kind, off) in plans[bi]:
                if phase == 0 and kind in ('L', 'A'):
                    push(pltpu.async_copy(
                        wnd.at[pl.ds(off, n)],
                        out_hbm.at[row, pl.ds(dst_j, n)], sem))
                elif phase == 1 and kind in ('B', 'R'):
                    push(pltpu.async_copy(
                        wnd.at[pl.ds(off, n)],
                        out_hbm.at[row, pl.ds(dst_j, n)], sem))
        if phase == 0:
            @pl.when(ri == 0)
            def _():
                descs0 = []

                def push0(d):
                    descs0.append(d)
                    if len(descs0) > MAXQ:
                        descs0.pop(0).wait()

                full = L // CROWS
                for t in range(full):
                    push0(pltpu.async_copy(
                        cbuf.at[pl.ds(0, CROWS)],
                        out_hbm.at[0, pl.ds(t * CROWS, CROWS)], sem))
                rem = L - full * CROWS
                push0(pltpu.async_copy(
                    cbuf.at[pl.ds(0, rem)],
                    out_hbm.at[0, pl.ds(full * CROWS, rem)], sem))
                for d in descs0:
                    d.wait()
        drain()


@jax.jit
def kernel(length_q, length_k, embeddings_table_v, embeddings_table_h):
    zq = (jnp.asarray(length_q) - L).astype(jnp.int32)
    zk = (jnp.asarray(length_k) - L).astype(jnp.int32)
    t = jnp.arange(32, dtype=jnp.int32)
    evi = jnp.where(t < 29, t + 1, 0) + zq
    ehi = jnp.where(t < 29, t + 1, 0) + zk
    tv128 = jnp.pad(embeddings_table_v, ((0, 0), (0, D)))
    th128 = jnp.pad(embeddings_table_h, ((0, 0), (0, D)))

    mesh = plsc.VectorSubcoreMesh(core_axis_name="c", subcore_axis_name="s")
    run = pl.kernel(
        _body,
        out_type=jax.ShapeDtypeStruct((L, L, D), jnp.float32),
        mesh=mesh,
        compiler_params=pltpu.CompilerParams(use_tc_tiling_on_sc=True),
        scratch_types=[
            pltpu.VMEM((32,), jnp.int32),
            pltpu.VMEM((32,), jnp.int32),
            pltpu.VMEM((32, 2 * D), jnp.float32),
            pltpu.VMEM((32, 2 * D), jnp.float32),
            pltpu.VMEM((HALF, D), jnp.float32),
            pltpu.VMEM((NHEAD * 8, D), jnp.float32),
            pltpu.VMEM((CROWS, D), jnp.float32),
            pltpu.SemaphoreType.DMA,
        ],
    )
    return run(tv128, th128, evi, ehi)
